# baseline (speedup 1.0000x reference)
import functools

import jax
import jax.numpy as jnp
from jax import lax
from jax.experimental import pallas as pl
from jax.experimental.pallas import tpu as pltpu
from jax.experimental.pallas import tpu_sc as plsc

D_MODEL = 128
SEQ_LEN = 512
VOCAB = 100
BATCH = 1024

NUM_CORES = 2
NUM_SUBCORES = 16
NUM_WORKERS = NUM_CORES * NUM_SUBCORES
SENT_PER_WORKER = BATCH // NUM_WORKERS

VPAD = 104
NPHASE = 32
PCH = SEQ_LEN // NPHASE
ESP_ROWS = PCH * VPAD
EB_ROWS = VPAD
NBUF = 4
NGRP = SENT_PER_WORKER // NBUF


def _positional_encoding():
    index = jnp.arange(0, D_MODEL, 2).astype(jnp.float32)
    denominator = jnp.power(10000.0, index / D_MODEL)
    position = jnp.arange(SEQ_LEN, dtype=jnp.float32)[:, None]
    even = jnp.sin(position / denominator)
    odd = jnp.cos(position / denominator)
    return jnp.stack((even, odd), axis=2).reshape(SEQ_LEN, D_MODEL)


def _sc_kernel(x2, table, pos):
    mesh = plsc.VectorSubcoreMesh(
        core_axis_name="c", subcore_axis_name="s",
        num_cores=NUM_CORES, num_subcores=NUM_SUBCORES)

    @functools.partial(
        pl.kernel,
        out_type=jax.ShapeDtypeStruct((BATCH, NPHASE, PCH, D_MODEL),
                                      jnp.float32),
        mesh=mesh,
        scratch_types=[
            pltpu.VMEM((SENT_PER_WORKER, SEQ_LEN), jnp.int32),
            pltpu.VMEM((NPHASE, SENT_PER_WORKER * PCH), jnp.int32),
            pltpu.VMEM((VOCAB, D_MODEL), jnp.float32),
            pltpu.VMEM((NPHASE, D_MODEL), jnp.float32),
            pltpu.VMEM((32,), jnp.int32),
            pltpu.VMEM((2, EB_ROWS, D_MODEL), jnp.float32),
            pltpu.VMEM((NBUF, PCH, D_MODEL), jnp.float32),
            pltpu.VMEM_SHARED((ESP_ROWS, D_MODEL), jnp.float32),
            pltpu.VMEM_SHARED((ESP_ROWS, D_MODEL), jnp.float32),
        ] + [pltpu.SemaphoreType.DMA] * (2 * NBUF + 3),
    )
    def k(x_ref, tab_ref, pos_ref, out_ref, idx_all, comb2, table_v, pos_my,
          lidx_v, ebuf, bufs, e_sp0, e_sp1, *sems):
        gsems = sems[:NBUF]
        ssems = sems[NBUF:2 * NBUF]
        edsems = sems[2 * NBUF:2 * NBUF + 2]
        psem = sems[2 * NBUF + 2]
        e_sps = (e_sp0, e_sp1)
        sid = lax.axis_index("s")
        wid = sid * NUM_CORES + lax.axis_index("c")
        base = wid * SENT_PER_WORKER

        pltpu.sync_copy(tab_ref, table_v)

        iota = lax.iota(jnp.int32, 16)
        for r in range(2):
            lidx_v[pl.ds(r * 16, 16)] = (iota + 16 * r) * PCH + sid
        pltpu.async_copy(pos_ref.at[lidx_v], pos_my, psem)

        pltpu.sync_copy(x_ref.at[pl.ds(base, SENT_PER_WORKER)], idx_all)

        offs = [(iota + 16 * r) * VPAD for r in range(PCH // 16)]

        def combi(b, carry):
            for ph in range(NPHASE):
                for r in range(PCH // 16):
                    comb2[ph, pl.ds(b * PCH + r * 16, 16)] = (
                        idx_all[b, pl.ds(ph * PCH + r * 16, 16)] + offs[r])
            return carry

        lax.fori_loop(0, SENT_PER_WORKER, combi, 0)

        pltpu.make_async_copy(pos_ref.at[lidx_v], pos_my, psem).wait()

        def compute_e(ph, par):
            p0 = [pos_my[ph, pl.ds(16 * j, 16)] for j in range(8)]

            def vrow(v, carry):
                for j in range(8):
                    sl = pl.ds(16 * j, 16)
                    ebuf[par, v, sl] = table_v[v, sl] + p0[j]
                return carry

            lax.fori_loop(0, VOCAB, vrow, 0)

        def fire_edma(par):
            pltpu.async_copy(
                ebuf.at[par], e_sps[par].at[pl.ds(EB_ROWS * sid, EB_ROWS)],
                edsems[par])

        def wait_edma(par):
            pltpu.make_async_copy(
                ebuf.at[par], e_sps[par].at[pl.ds(EB_ROWS * sid, EB_ROWS)],
                edsems[par]).wait()

        compute_e(0, 0)
        fire_edma(0)
        compute_e(1, 1)
        fire_edma(1)

        def fire_gather(e_sp, ph, b, j):
            pltpu.async_copy(
                e_sp.at[comb2.at[ph, pl.ds(b * PCH, PCH)]], bufs.at[j],
                gsems[j])

        def wait_gather(e_sp, ph, b, j):
            pltpu.make_async_copy(
                e_sp.at[comb2.at[ph, pl.ds(b * PCH, PCH)]], bufs.at[j],
                gsems[j]).wait()

        def fire_scatter(ph, b, j):
            pltpu.async_copy(bufs.at[j], out_ref.at[base + b, ph], ssems[j])

        def wait_scatter(ph, b, j):
            pltpu.make_async_copy(
                bufs.at[j], out_ref.at[base + b, ph], ssems[j]).wait()

        def phase_pair(pp, carry):
            for half in range(2):
                ph = 2 * pp + half
                e_sp = e_sps[half]
                wait_edma(half)
                plsc.subcore_barrier()

                for j in range(NBUF):
                    if half == 0:
                        @pl.when(pp > 0)
                        def _w(ph=ph, j=j):
                            wait_scatter(ph - 1,
                                         SENT_PER_WORKER - NBUF + j, j)
                    else:
                        wait_scatter(ph - 1, SENT_PER_WORKER - NBUF + j, j)
                    fire_gather(e_sp, ph, j, j)

                @pl.when(pp < (NPHASE // 2) - 1)
                def _build(ph=ph, half=half):
                    compute_e(ph + 2, half)

                def group(g, carry):
                    for j in range(NBUF):
                        wait_gather(e_sp, ph, g * NBUF + j, j)
                        fire_scatter(ph, g * NBUF + j, j)

                    @pl.when(g < NGRP - 1)
                    def _next(g=g):
                        for j in range(NBUF):
                            wait_scatter(ph, g * NBUF + j, j)
                            fire_gather(e_sp, ph, (g + 1) * NBUF + j, j)
                    return carry

                lax.fori_loop(0, NGRP, group, 0)

                plsc.subcore_barrier()

                @pl.when(pp < (NPHASE // 2) - 1)
                def _restage(half=half):
                    fire_edma(half)
            return carry

        lax.fori_loop(0, NPHASE // 2, phase_pair, 0)

        for j in range(NBUF):
            wait_scatter(NPHASE - 1, SENT_PER_WORKER - NBUF + j, j)

    return k(x2, table, pos)


def kernel(x, table):
    pos = _positional_encoding()
    x2 = x.astype(jnp.int32)
    out4 = _sc_kernel(x2, table, pos)
    return out4.reshape(BATCH, SEQ_LEN, D_MODEL)


# speedup vs baseline: 1.1008x; 1.1008x over previous
import functools

import jax
import jax.numpy as jnp
from jax import lax
from jax.experimental import pallas as pl
from jax.experimental.pallas import tpu as pltpu
from jax.experimental.pallas import tpu_sc as plsc

D_MODEL = 128
SEQ_LEN = 512
VOCAB = 100
BATCH = 1024

NUM_CORES = 2
NUM_SUBCORES = 16
NUM_WORKERS = NUM_CORES * NUM_SUBCORES
SENT_PER_WORKER = BATCH // NUM_WORKERS


def _positional_encoding():
    index = jnp.arange(0, D_MODEL, 2).astype(jnp.float32)
    denominator = jnp.power(10000.0, index / D_MODEL)
    position = jnp.arange(SEQ_LEN, dtype=jnp.float32)[:, None]
    even = jnp.sin(position / denominator)
    odd = jnp.cos(position / denominator)
    return jnp.stack((even, odd), axis=2).reshape(SEQ_LEN, D_MODEL)


VPAD = 104


def _build_expanded_table(table_pad, pos):
    lblk = 32

    def body(tab_ref, pos_ref, o_ref):
        o_ref[...] = (
            pos_ref[...][:, None, :] + tab_ref[...][None, :, :]
        ).reshape(lblk * VPAD, D_MODEL)

    return pl.pallas_call(
        body,
        grid=(SEQ_LEN // lblk,),
        in_specs=[
            pl.BlockSpec((VPAD, D_MODEL), lambda i: (0, 0)),
            pl.BlockSpec((lblk, D_MODEL), lambda i: (i, 0)),
        ],
        out_specs=pl.BlockSpec((lblk * VPAD, D_MODEL), lambda i: (i, 0)),
        out_shape=jax.ShapeDtypeStruct((SEQ_LEN * VPAD, D_MODEL),
                                       jnp.float32),
    )(table_pad, pos)


NPHASE = 16
PCH = SEQ_LEN // NPHASE
ESP_ROWS = PCH * VPAD
NBUF = 8


def _sc_gather(x2, e2, offs):
    mesh = plsc.VectorSubcoreMesh(
        core_axis_name="c", subcore_axis_name="s",
        num_cores=NUM_CORES, num_subcores=NUM_SUBCORES)

    @functools.partial(
        pl.kernel,
        out_type=jax.ShapeDtypeStruct((BATCH, NPHASE, PCH, D_MODEL),
                                      jnp.float32),
        mesh=mesh,
        scratch_types=[
            pltpu.VMEM((SENT_PER_WORKER, SEQ_LEN), jnp.int32),
            pltpu.VMEM((SENT_PER_WORKER, SEQ_LEN), jnp.int32),
            pltpu.VMEM((SEQ_LEN,), jnp.int32),
            pltpu.VMEM((NBUF, PCH, D_MODEL), jnp.float32),
            pltpu.VMEM_SHARED((ESP_ROWS, D_MODEL), jnp.float32),
            pltpu.VMEM_SHARED((ESP_ROWS, D_MODEL), jnp.float32),
        ] + [pltpu.SemaphoreType.DMA] * (2 * NBUF + 2),
    )
    def k(x_ref, e_ref, offs_ref, out_ref, idx_all, comb_all, offs_v, bufs,
          e_sp0, e_sp1, *sems):
        gsems = sems[:NBUF]
        ssems = sems[NBUF:2 * NBUF]
        stgsems = sems[2 * NBUF:]
        e_sps = (e_sp0, e_sp1)
        sid = lax.axis_index("s")
        wid = sid * NUM_CORES + lax.axis_index("c")
        base = wid * SENT_PER_WORKER

        @pl.when(sid == 0)
        def _stage01():
            pltpu.async_copy(
                e_ref.at[pl.ds(0, ESP_ROWS)], e_sps[0], stgsems[0])
            pltpu.async_copy(
                e_ref.at[pl.ds(ESP_ROWS, ESP_ROWS)], e_sps[1], stgsems[1])

        pltpu.sync_copy(offs_ref, offs_v)
        pltpu.sync_copy(x_ref.at[pl.ds(base, SENT_PER_WORKER)], idx_all)

        def combi(b, carry):
            for r in range(SEQ_LEN // 16):
                sl = pl.ds(r * 16, 16)
                comb_all[b, sl] = idx_all[b, sl] + offs_v[sl]
            return carry

        lax.fori_loop(0, SENT_PER_WORKER, combi, 0)

        for ph in range(NPHASE):
            e_sp = e_sps[ph % 2]

            @pl.when(sid == 0)
            def _wait_stage():
                pltpu.make_async_copy(
                    e_ref.at[pl.ds(ph * ESP_ROWS, ESP_ROWS)], e_sp,
                    stgsems[ph % 2]).wait()

            plsc.subcore_barrier()

            def group(g, carry):
                for j in range(NBUF):
                    b = g * NBUF + j
                    bb = base + b
                    if ph == 0:
                        @pl.when(g > 0)
                        def _wait_prev():
                            pltpu.make_async_copy(
                                bufs.at[j], out_ref.at[bb, ph],
                                ssems[j]).wait()
                    else:
                        pltpu.make_async_copy(
                            bufs.at[j], out_ref.at[bb, ph], ssems[j]).wait()
                    pltpu.async_copy(
                        e_sp.at[comb_all.at[b, pl.ds(ph * PCH, PCH)]],
                        bufs.at[j], gsems[j])
                for j in range(NBUF):
                    b = g * NBUF + j
                    bb = base + b
                    pltpu.make_async_copy(
                        e_sp.at[comb_all.at[b, pl.ds(ph * PCH, PCH)]],
                        bufs.at[j], gsems[j]).wait()
                    pltpu.async_copy(bufs.at[j], out_ref.at[bb, ph],
                                     ssems[j])
                return carry

            lax.fori_loop(0, SENT_PER_WORKER // NBUF, group, 0)

            plsc.subcore_barrier()
            if ph + 2 < NPHASE:
                @pl.when(sid == 0)
                def _stage_next():
                    pltpu.async_copy(
                        e_ref.at[pl.ds((ph + 2) * ESP_ROWS, ESP_ROWS)],
                        e_sps[ph % 2], stgsems[ph % 2])

        for j in range(NBUF):
            bb = base + SENT_PER_WORKER - NBUF + j
            pltpu.make_async_copy(
                bufs.at[j], out_ref.at[bb, NPHASE - 1], ssems[j]).wait()

    return k(x2, e2, offs)


def kernel(x, table):
    pos = _positional_encoding()
    table_pad = jnp.pad(table, ((0, VPAD - VOCAB), (0, 0)))
    e2 = _build_expanded_table(table_pad, pos)
    x2 = x.astype(jnp.int32)
    offs = (jnp.arange(SEQ_LEN, dtype=jnp.int32) % PCH) * VPAD
    out4 = _sc_gather(x2, e2, offs)
    return out4.reshape(BATCH, SEQ_LEN, D_MODEL)
